# baseline (speedup 1.0000x reference)
import functools

import jax
import jax.numpy as jnp
from jax import lax
from jax.experimental import pallas as pl
from jax.experimental.pallas import tpu as pltpu
from jax.experimental.pallas import tpu_sc as plsc

B, S, D, P, TOPK = 1024, 50, 128, 1024, 20
NEG_BIG = -3.0e38
MASK_FILL = -1.0e7
ROWS = B * S
NW = 32
ROWS_PER_W = ROWS // NW
RB = 32
NBLK = ROWS_PER_W // RB


def _matmul_kernel(ss_ref, proto_ref, sim_ref, *, bb):
    ss = ss_ref[...].reshape(bb * S, D)
    sim_ref[...] = jax.lax.dot_general(
        ss, proto_ref[...],
        dimension_numbers=(((1,), (1,)), ((), ())),
        preferred_element_type=jnp.float32,
    )


_GDN = lax.GatherDimensionNumbers(
    offset_dims=(), collapsed_slice_dims=(0,), start_index_map=(0,))


def _perm(x, idx):
    return lax.gather(x, idx[:, None], dimension_numbers=_GDN,
                      slice_sizes=(1,),
                      mode=lax.GatherScatterMode.PROMISE_IN_BOUNDS)


def _vmax_splat(x):
    lanes = lax.iota(jnp.int32, 16)
    for d in (1, 2, 4, 8):
        x = jnp.maximum(x, _perm(x, lanes ^ d))
    return x


def _vmin_splat(x):
    lanes = lax.iota(jnp.int32, 16)
    for d in (1, 2, 4, 8):
        x = jnp.minimum(x, _perm(x, lanes ^ d))
    return x


def _sc_threshold(sim_hbm, out_hbm, buf, outbuf, sem):
    wid = lax.axis_index("s") * 2 + lax.axis_index("c")
    base = wid * ROWS_PER_W
    lanes = lax.iota(jnp.int32, 16)
    negv = jnp.full((16,), NEG_BIG, jnp.float32)

    def block_body(blk, _):
        r0 = base + blk * RB
        pltpu.async_copy(sim_hbm.at[pl.ds(r0 * P, RB * P)], buf, sem).wait()

        def row_chain(h):
            def row_body(i, acc):
                r = h * 16 + i
                rbase = r * 1024
                cm = buf[pl.ds(rbase, 16)]
                for v in range(1, 64):
                    cm = jnp.maximum(cm, buf[pl.ds(rbase + 16 * v, 16)])

                def pop(_, carry):
                    cm, _t = carry
                    gv = _vmax_splat(cm)
                    popm = cm == gv
                    l = _vmin_splat(jnp.where(popm, lanes,
                                              jnp.full((16,), 16,
                                                       jnp.int32)))
                    newm = negv
                    rbase_v = jnp.full((16,), rbase, jnp.int32)
                    for k in range(4):
                        idx = rbase_v + l + 16 * (lanes + 16 * k)
                        vals = plsc.load_gather(buf, [idx])
                        vals = jnp.where(vals < gv, vals, negv)
                        newm = jnp.maximum(newm, vals)
                    nm = _vmax_splat(newm)
                    cm = jnp.where(popm, nm, cm)
                    return (cm, gv)

                _cm, t = lax.fori_loop(0, TOPK, pop,
                                       (cm, jnp.zeros((16,), jnp.float32)))
                acc = jnp.where(lanes == jnp.full((16,), i, jnp.int32),
                                t, acc)
                return acc

            acc = lax.fori_loop(0, 16, row_body,
                                jnp.zeros((16,), jnp.float32))
            outbuf[pl.ds(16 * h, 16)] = acc

        row_chain(0)
        row_chain(1)
        pltpu.sync_copy(outbuf, out_hbm.at[pl.ds(r0, RB)])
        return 0

    lax.fori_loop(0, NBLK, block_body, 0)


def _finish_kernel(sim_ref, t_ref, proto_ref, bool_ref, emb_ref, *, bb):
    sim = sim_ref[...]
    sim3 = sim.reshape(bb, S, P)
    t3 = t_ref[...].reshape(bb, S, 1)
    in_topk = (sim3 >= t3).astype(jnp.float32)
    cnt = jnp.sum(in_topk, axis=1)
    mask = cnt >= jnp.float32(S)
    mean = jnp.mean(sim3, axis=1)
    masked = jnp.where(mask, mean, jnp.float32(MASK_FILL))
    m = jnp.max(masked, axis=-1, keepdims=True)
    e = jnp.exp(masked - m)
    dist = e / jnp.sum(e, axis=-1, keepdims=True)
    emb = jax.lax.dot_general(
        dist, proto_ref[...],
        dimension_numbers=(((1,), (0,)), ((), ())),
        preferred_element_type=jnp.float32,
    )
    norm = jnp.sqrt(jnp.sum(emb * emb, axis=-1, keepdims=True))
    emb = emb / jnp.maximum(norm, jnp.float32(1e-12))
    bool_ref[...] = mask
    emb_ref[...] = emb


def kernel(support_sets, proto_embs):
    bb = 16
    grid = (B // bb,)

    sim = pl.pallas_call(
        functools.partial(_matmul_kernel, bb=bb),
        grid=grid,
        in_specs=[
            pl.BlockSpec((bb, S, D), lambda i: (i, 0, 0)),
            pl.BlockSpec((P, D), lambda i: (0, 0)),
        ],
        out_specs=pl.BlockSpec((bb * S, P), lambda i: (i, 0)),
        out_shape=jax.ShapeDtypeStruct((ROWS, P), jnp.float32),
    )(support_sets, proto_embs)

    mesh = plsc.VectorSubcoreMesh(core_axis_name="c", subcore_axis_name="s")
    sc_fn = pl.kernel(
        _sc_threshold,
        mesh=mesh,
        compiler_params=pltpu.CompilerParams(needs_layout_passes=False),
        out_type=jax.ShapeDtypeStruct((ROWS,), jnp.float32),
        scratch_types=[
            pltpu.VMEM((RB * P,), jnp.float32),
            pltpu.VMEM((RB,), jnp.float32),
            pltpu.SemaphoreType.DMA,
        ],
    )
    t = sc_fn(sim.reshape(-1))

    out_bool, out_emb = pl.pallas_call(
        functools.partial(_finish_kernel, bb=bb),
        grid=grid,
        in_specs=[
            pl.BlockSpec((bb * S, P), lambda i: (i, 0)),
            pl.BlockSpec((bb * S, 1), lambda i: (i, 0)),
            pl.BlockSpec((P, D), lambda i: (0, 0)),
        ],
        out_specs=[
            pl.BlockSpec((bb, P), lambda i: (i, 0)),
            pl.BlockSpec((bb, D), lambda i: (i, 0)),
        ],
        out_shape=[
            jax.ShapeDtypeStruct((B, P), jnp.bool_),
            jax.ShapeDtypeStruct((B, D), jnp.float32),
        ],
    )(sim, t.reshape(ROWS, 1), proto_embs)
    return out_bool, out_emb


# speedup vs baseline: 4.4876x; 4.4876x over previous
import functools

import jax
import jax.numpy as jnp
from jax.experimental import pallas as pl
from jax.experimental.pallas import tpu as pltpu

B, S, D, P, TOPK = 1024, 50, 128, 1024, 20
NEG_BIG = -3.0e38
MASK_FILL = -1.0e7

_SORT8 = [
    (0, 1), (2, 3), (4, 5), (6, 7),
    (0, 2), (1, 3), (4, 6), (5, 7),
    (1, 2), (5, 6),
    (0, 4), (1, 5), (2, 6), (3, 7),
    (2, 4), (3, 5),
    (1, 2), (3, 4), (5, 6),
]


def _sorted_cols(sim):
    lvl = [sim[:, 128 * j:128 * (j + 1)] for j in range(8)]
    for i, j in _SORT8:
        hi = jnp.maximum(lvl[i], lvl[j])
        lo = jnp.minimum(lvl[i], lvl[j])
        lvl[i], lvl[j] = hi, lo
    return lvl


def _pop_extract(lvl):
    nl = len(lvl)
    thresh = None
    for it in range(TOPK):
        thresh = jnp.max(lvl[0], axis=-1, keepdims=True)
        if it < TOPK - 1:
            popm = lvl[0] == thresh
            for j in range(nl - 1):
                lvl[j] = jnp.where(popm, lvl[j + 1], lvl[j])
            lvl[nl - 1] = jnp.where(popm, NEG_BIG, lvl[nl - 1])
    return thresh, lvl[0]


def _fused_kernel(ss_ref, proto_ref, bool_ref, emb_ref, t_ref, *, bb):
    rows = bb * S
    ss = ss_ref[...].reshape(rows, D)
    proto = proto_ref[...]
    sim = jax.lax.dot_general(
        ss, proto,
        dimension_numbers=(((1,), (1,)), ((), ())),
        preferred_element_type=jnp.float32,
    )

    lvl = _sorted_cols(sim)
    thresh, heads = _pop_extract(lvl[:5])
    t_ref[...] = thresh
    need_full = jnp.min(heads) < jnp.float32(-1.0e37)

    @pl.when(need_full)
    def _fallback():
        full_t, _ = _pop_extract(_sorted_cols(sim))
        t_ref[...] = full_t

    sim3 = sim.reshape(bb, S, P)
    t3 = t_ref[...].reshape(bb, S, 1)
    in_topk = (sim3 >= t3).astype(jnp.float32)
    cnt = jnp.sum(in_topk, axis=1)
    mask = cnt >= jnp.float32(S)
    mean = jnp.mean(sim3, axis=1)
    masked = jnp.where(mask, mean, jnp.float32(MASK_FILL))
    m = jnp.max(masked, axis=-1, keepdims=True)
    e = jnp.exp(masked - m)
    dist = e / jnp.sum(e, axis=-1, keepdims=True)
    emb = jax.lax.dot_general(
        dist, proto,
        dimension_numbers=(((1,), (0,)), ((), ())),
        preferred_element_type=jnp.float32,
    )
    norm = jnp.sqrt(jnp.sum(emb * emb, axis=-1, keepdims=True))
    emb = emb / jnp.maximum(norm, jnp.float32(1e-12))
    bool_ref[...] = mask
    emb_ref[...] = emb


def kernel(support_sets, proto_embs):
    bb = 16
    grid = (B // bb,)
    f = functools.partial(_fused_kernel, bb=bb)
    out_bool, out_emb = pl.pallas_call(
        f,
        grid=grid,
        in_specs=[
            pl.BlockSpec((bb, S, D), lambda i: (i, 0, 0)),
            pl.BlockSpec((P, D), lambda i: (0, 0)),
        ],
        out_specs=[
            pl.BlockSpec((bb, P), lambda i: (i, 0)),
            pl.BlockSpec((bb, D), lambda i: (i, 0)),
        ],
        out_shape=[
            jax.ShapeDtypeStruct((B, P), jnp.bool_),
            jax.ShapeDtypeStruct((B, D), jnp.float32),
        ],
        scratch_shapes=[pltpu.VMEM((bb * S, 1), jnp.float32)],
    )(support_sets, proto_embs)
    return out_bool, out_emb
